# baseline (speedup 1.0000x reference)
import functools

import jax
import jax.numpy as jnp
from jax import lax
from jax.experimental import pallas as pl
from jax.experimental.pallas import tpu as pltpu
from jax.experimental.pallas import tpu_sc as plsc

NC = 2
NS = 16
NW = NC * NS
LANES = 16
CHUNK = 160
WINDOWS = ((0, 128), (128, 32))


def _emb_kernel(tok_hbm, comb_hbm, idx_hbm, cidx_hbm, out_hbm,
                idx_v0, idx_v1, cidx_v0, cidx_v1, tok_v0, tok_v1, comb_v,
                semg0, semg1, semo0, semo1, semi0, semi1):
  n_rows = out_hbm.shape[0]
  emb = comb_hbm.shape[1]
  rows_per_tile = n_rows // NW
  n_chunks = rows_per_tile // CHUNK
  wid = lax.axis_index("s") * NC + lax.axis_index("c")
  row0 = wid * rows_per_tile
  lane = lax.iota(jnp.int32, LANES)

  pltpu.sync_copy(comb_hbm, comb_v)

  def fire_idx(k, iv, cv, sem):
    rb = row0 + k * CHUNK
    pltpu.async_copy(idx_hbm.at[pl.ds(rb, CHUNK)], iv, sem)
    pltpu.async_copy(cidx_hbm.at[pl.ds(rb, CHUNK)], cv, sem)

  def drain_idx(iv, cv, sem):
    pltpu.make_async_copy(idx_hbm.at[pl.ds(0, CHUNK)], iv, sem).wait()
    pltpu.make_async_copy(cidx_hbm.at[pl.ds(0, CHUNK)], cv, sem).wait()

  def fire_g(iv, tv, sem):
    for off, ln in WINDOWS:
      w = pl.ds(off, ln)
      pltpu.async_copy(tok_hbm.at[iv.at[w]], tv.at[w], sem)

  def drain_g(iv, tv, sem):
    for off, ln in WINDOWS:
      w = pl.ds(off, ln)
      pltpu.make_async_copy(tok_hbm.at[iv.at[w]], tv.at[w], sem).wait()

  def fire_out(k, tv, sem):
    rb = row0 + k * CHUNK
    pltpu.async_copy(tv, out_hbm.at[pl.ds(rb, CHUNK)], sem)

  def drain_out(tv, sem):
    pltpu.make_async_copy(tv, out_hbm.at[pl.ds(0, CHUNK)], sem).wait()

  def tec_add(tv, cv):
    @pl.loop(0, CHUNK, step=LANES)
    def _blk(r0):
      cidx16 = cv[pl.ds(r0, LANES)]
      rowv = jnp.broadcast_to(r0, (LANES,)).astype(jnp.int32) + lane
      colv = jnp.zeros((LANES,), jnp.int32)
      for c in range(emb):
        tokvals = plsc.load_gather(tv, [rowv, colv])
        combvals = plsc.load_gather(comb_v, [cidx16, colv])
        plsc.store_scatter(tv, [rowv, colv], tokvals + combvals)
        if c + 1 < emb:
          colv = colv + 1

  fire_idx(0, idx_v0, cidx_v0, semi0)
  fire_idx(1, idx_v1, cidx_v1, semi1)
  drain_idx(idx_v0, cidx_v0, semi0)
  fire_g(idx_v0, tok_v0, semg0)

  drain_g(idx_v0, tok_v0, semg0)
  drain_idx(idx_v1, cidx_v1, semi1)
  fire_g(idx_v1, tok_v1, semg1)
  tec_add(tok_v0, cidx_v0)
  fire_idx(2, idx_v0, cidx_v0, semi0)
  fire_out(0, tok_v0, semo0)

  @pl.loop(1, n_chunks - 2, step=2)
  def _pair(k):
    drain_g(idx_v1, tok_v1, semg1)
    drain_out(tok_v0, semo0)
    drain_idx(idx_v0, cidx_v0, semi0)
    fire_g(idx_v0, tok_v0, semg0)
    tec_add(tok_v1, cidx_v1)
    fire_idx(k + 2, idx_v1, cidx_v1, semi1)
    fire_out(k, tok_v1, semo1)

    drain_g(idx_v0, tok_v0, semg0)
    drain_out(tok_v1, semo1)
    drain_idx(idx_v1, cidx_v1, semi1)
    fire_g(idx_v1, tok_v1, semg1)
    tec_add(tok_v0, cidx_v0)

    @pl.when(k + 3 < n_chunks)
    def _():
      fire_idx(k + 3, idx_v0, cidx_v0, semi0)

    fire_out(k + 1, tok_v0, semo0)

  drain_g(idx_v1, tok_v1, semg1)
  drain_out(tok_v0, semo0)
  tec_add(tok_v1, cidx_v1)
  fire_out(n_chunks - 1, tok_v1, semo1)
  drain_out(tok_v1, semo1)


def kernel(x, segment_label, token_table, position_table, segment_table):
  batch, seq = x.shape
  emb = token_table.shape[1]
  n = batch * seq

  table128 = jnp.pad(token_table, ((0, 0), (0, 128 - emb)))

  nseg = segment_table.shape[0]
  combined = (position_table[:seq, None, :]
              + segment_table[None, :, :]).reshape(seq * nseg, emb)

  idx = x.reshape(n).astype(jnp.int32)
  cidx = (jnp.arange(seq, dtype=jnp.int32)[None, :] * nseg
          + segment_label.astype(jnp.int32)).reshape(n)

  mesh = plsc.VectorSubcoreMesh(core_axis_name="c", subcore_axis_name="s",
                                num_cores=NC, num_subcores=NS)
  run = pl.kernel(
      _emb_kernel,
      out_type=jax.ShapeDtypeStruct((n, 128), jnp.float32),
      mesh=mesh,
      scratch_types=[
          pltpu.VMEM((CHUNK,), jnp.int32),
          pltpu.VMEM((CHUNK,), jnp.int32),
          pltpu.VMEM((CHUNK,), jnp.int32),
          pltpu.VMEM((CHUNK,), jnp.int32),
          pltpu.VMEM((CHUNK, 128), jnp.float32),
          pltpu.VMEM((CHUNK, 128), jnp.float32),
          pltpu.VMEM((seq * segment_table.shape[0], emb), jnp.float32),
          pltpu.SemaphoreType.DMA,
          pltpu.SemaphoreType.DMA,
          pltpu.SemaphoreType.DMA,
          pltpu.SemaphoreType.DMA,
          pltpu.SemaphoreType.DMA,
          pltpu.SemaphoreType.DMA,
      ],
      compiler_params=pltpu.CompilerParams(use_tc_tiling_on_sc=True,
                                           needs_layout_passes=False),
  )
  out128 = run(table128, combined, idx, cidx)
  return out128.reshape(batch, seq, 128)[:, :, :emb]


# speedup vs baseline: 2.7944x; 2.7944x over previous
import functools

import jax
import jax.numpy as jnp
from jax import lax
from jax.experimental import pallas as pl
from jax.experimental.pallas import tpu as pltpu
from jax.experimental.pallas import tpu_sc as plsc

NC = 2
NS = 16
NW = NC * NS
CHUNK = 320
WINDOWS = ((0, 128), (128, 128), (256, 64))


def _emb_kernel(tok_hbm, comb_hbm, idx_hbm, cidx_hbm, out_hbm,
                idx_v0, idx_v1, cidx_v0, cidx_v1, tok_v0, tok_v1,
                semg0, semg1, semc0, semc1, semo0, semo1, semi0, semi1):
  n_rows = out_hbm.shape[0]
  rows_per_tile = n_rows // NW
  n_chunks = rows_per_tile // CHUNK
  wid = lax.axis_index("s") * NC + lax.axis_index("c")
  row0 = wid * rows_per_tile

  def fire_idx(k, iv, cv, sem):
    rb = row0 + k * CHUNK
    pltpu.async_copy(idx_hbm.at[pl.ds(rb, CHUNK)], iv, sem)
    pltpu.async_copy(cidx_hbm.at[pl.ds(rb, CHUNK)], cv, sem)

  def drain_idx(iv, cv, sem):
    pltpu.make_async_copy(idx_hbm.at[pl.ds(0, CHUNK)], iv, sem).wait()
    pltpu.make_async_copy(cidx_hbm.at[pl.ds(0, CHUNK)], cv, sem).wait()

  def fire_g(iv, tv, sem):
    for off, ln in WINDOWS:
      w = pl.ds(off, ln)
      pltpu.async_copy(tok_hbm.at[iv.at[w]], tv.at[w], sem)

  def drain_g(iv, tv, sem):
    for off, ln in WINDOWS:
      w = pl.ds(off, ln)
      pltpu.make_async_copy(tok_hbm.at[iv.at[w]], tv.at[w], sem).wait()

  def fire_comb(cv, tv, sem):
    for off, ln in WINDOWS:
      w = pl.ds(off, ln)
      pltpu.async_copy(comb_hbm.at[cv.at[w]], tv.at[w], sem, add=True)

  def drain_comb(cv, tv, sem):
    for off, ln in WINDOWS:
      w = pl.ds(off, ln)
      pltpu.make_async_copy(comb_hbm.at[cv.at[w]], tv.at[w], sem).wait()

  def fire_out(k, tv, sem):
    rb = row0 + k * CHUNK
    pltpu.async_copy(tv, out_hbm.at[pl.ds(rb, CHUNK)], sem)

  def drain_out(tv, sem):
    pltpu.make_async_copy(tv, out_hbm.at[pl.ds(0, CHUNK)], sem).wait()

  fire_idx(0, idx_v0, cidx_v0, semi0)
  fire_idx(1, idx_v1, cidx_v1, semi1)
  drain_idx(idx_v0, cidx_v0, semi0)
  fire_g(idx_v0, tok_v0, semg0)

  drain_g(idx_v0, tok_v0, semg0)
  drain_idx(idx_v1, cidx_v1, semi1)
  fire_g(idx_v1, tok_v1, semg1)
  fire_comb(cidx_v0, tok_v0, semc0)
  drain_comb(cidx_v0, tok_v0, semc0)
  fire_idx(2, idx_v0, cidx_v0, semi0)
  fire_out(0, tok_v0, semo0)

  @pl.loop(1, n_chunks - 2, step=2)
  def _pair(k):
    drain_g(idx_v1, tok_v1, semg1)
    drain_out(tok_v0, semo0)
    drain_idx(idx_v0, cidx_v0, semi0)
    fire_g(idx_v0, tok_v0, semg0)
    fire_comb(cidx_v1, tok_v1, semc1)
    drain_comb(cidx_v1, tok_v1, semc1)
    fire_idx(k + 2, idx_v1, cidx_v1, semi1)
    fire_out(k, tok_v1, semo1)

    drain_g(idx_v0, tok_v0, semg0)
    drain_out(tok_v1, semo1)
    drain_idx(idx_v1, cidx_v1, semi1)
    fire_g(idx_v1, tok_v1, semg1)
    fire_comb(cidx_v0, tok_v0, semc0)
    drain_comb(cidx_v0, tok_v0, semc0)

    @pl.when(k + 3 < n_chunks)
    def _():
      fire_idx(k + 3, idx_v0, cidx_v0, semi0)

    fire_out(k + 1, tok_v0, semo0)

  drain_g(idx_v1, tok_v1, semg1)
  drain_out(tok_v0, semo0)
  fire_comb(cidx_v1, tok_v1, semc1)
  drain_comb(cidx_v1, tok_v1, semc1)
  fire_out(n_chunks - 1, tok_v1, semo1)
  drain_out(tok_v1, semo1)


def kernel(x, segment_label, token_table, position_table, segment_table):
  batch, seq = x.shape
  emb = token_table.shape[1]
  n = batch * seq

  table128 = jnp.pad(token_table, ((0, 0), (0, 128 - emb)))

  nseg = segment_table.shape[0]
  combined = (position_table[:seq, None, :]
              + segment_table[None, :, :]).reshape(seq * nseg, emb)
  comb128 = jnp.pad(combined, ((0, 0), (0, 128 - emb)))

  idx = x.reshape(n).astype(jnp.int32)
  cidx = (jnp.arange(seq, dtype=jnp.int32)[None, :] * nseg
          + segment_label.astype(jnp.int32)).reshape(n)

  mesh = plsc.VectorSubcoreMesh(core_axis_name="c", subcore_axis_name="s",
                                num_cores=NC, num_subcores=NS)
  run = pl.kernel(
      _emb_kernel,
      out_type=jax.ShapeDtypeStruct((n, 128), jnp.float32),
      mesh=mesh,
      scratch_types=[
          pltpu.VMEM((CHUNK,), jnp.int32),
          pltpu.VMEM((CHUNK,), jnp.int32),
          pltpu.VMEM((CHUNK,), jnp.int32),
          pltpu.VMEM((CHUNK,), jnp.int32),
          pltpu.VMEM((CHUNK, 128), jnp.float32),
          pltpu.VMEM((CHUNK, 128), jnp.float32),
          pltpu.SemaphoreType.DMA,
          pltpu.SemaphoreType.DMA,
          pltpu.SemaphoreType.DMA,
          pltpu.SemaphoreType.DMA,
          pltpu.SemaphoreType.DMA,
          pltpu.SemaphoreType.DMA,
          pltpu.SemaphoreType.DMA,
          pltpu.SemaphoreType.DMA,
      ],
      compiler_params=pltpu.CompilerParams(use_tc_tiling_on_sc=True),
  )
  out128 = run(table128, comb128, idx, cidx)
  return out128.reshape(batch, seq, 128)[:, :, :emb]


# speedup vs baseline: 3.6439x; 1.3040x over previous
import functools

import jax
import jax.numpy as jnp
from jax import lax
from jax.experimental import pallas as pl
from jax.experimental.pallas import tpu as pltpu
from jax.experimental.pallas import tpu_sc as plsc

NC = 2
NS = 16
NW = NC * NS
CHUNK = 320
WINDOWS = ((0, 128), (128, 128), (256, 64))


def _emb_kernel(tok_hbm, comb_hbm, idx_hbm, cidx_hbm, out_hbm,
                idx_v0, idx_v1, cidx_v0, cidx_v1, tok_v0, tok_v1, comb_sh,
                semg0, semg1, semc0, semc1, semo0, semo1, semi0, semi1):
  n_rows = out_hbm.shape[0]
  rows_per_tile = n_rows // NW
  n_chunks = rows_per_tile // CHUNK
  wid = lax.axis_index("s") * NC + lax.axis_index("c")
  row0 = wid * rows_per_tile

  @pl.when(lax.axis_index("s") == 0)
  def _load_comb():
    pltpu.sync_copy(comb_hbm, comb_sh)

  plsc.subcore_barrier()

  def fire_idx(k, iv, cv, sem):
    rb = row0 + k * CHUNK
    pltpu.async_copy(idx_hbm.at[pl.ds(rb, CHUNK)], iv, sem)
    pltpu.async_copy(cidx_hbm.at[pl.ds(rb, CHUNK)], cv, sem)

  def drain_idx(iv, cv, sem):
    pltpu.make_async_copy(idx_hbm.at[pl.ds(0, CHUNK)], iv, sem).wait()
    pltpu.make_async_copy(cidx_hbm.at[pl.ds(0, CHUNK)], cv, sem).wait()

  def fire_g(iv, tv, sem):
    for off, ln in WINDOWS:
      w = pl.ds(off, ln)
      pltpu.async_copy(tok_hbm.at[iv.at[w]], tv.at[w], sem)

  def drain_g(iv, tv, sem):
    for off, ln in WINDOWS:
      w = pl.ds(off, ln)
      pltpu.make_async_copy(tok_hbm.at[iv.at[w]], tv.at[w], sem).wait()

  def fire_comb(cv, tv, sem):
    for off, ln in WINDOWS:
      w = pl.ds(off, ln)
      pltpu.async_copy(comb_sh.at[cv.at[w]], tv.at[w], sem, add=True)

  def drain_comb(cv, tv, sem):
    for off, ln in WINDOWS:
      w = pl.ds(off, ln)
      pltpu.make_async_copy(comb_sh.at[cv.at[w]], tv.at[w], sem).wait()

  def fire_out(k, tv, sem):
    rb = row0 + k * CHUNK
    pltpu.async_copy(tv, out_hbm.at[pl.ds(rb, CHUNK)], sem)

  def drain_out(tv, sem):
    pltpu.make_async_copy(tv, out_hbm.at[pl.ds(0, CHUNK)], sem).wait()

  fire_idx(0, idx_v0, cidx_v0, semi0)
  fire_idx(1, idx_v1, cidx_v1, semi1)
  drain_idx(idx_v0, cidx_v0, semi0)
  fire_g(idx_v0, tok_v0, semg0)

  drain_g(idx_v0, tok_v0, semg0)
  drain_idx(idx_v1, cidx_v1, semi1)
  fire_g(idx_v1, tok_v1, semg1)
  fire_comb(cidx_v0, tok_v0, semc0)
  drain_comb(cidx_v0, tok_v0, semc0)
  fire_idx(2, idx_v0, cidx_v0, semi0)
  fire_out(0, tok_v0, semo0)

  @pl.loop(1, n_chunks - 2, step=2)
  def _pair(k):
    drain_g(idx_v1, tok_v1, semg1)
    drain_out(tok_v0, semo0)
    drain_idx(idx_v0, cidx_v0, semi0)
    fire_g(idx_v0, tok_v0, semg0)
    fire_comb(cidx_v1, tok_v1, semc1)
    drain_comb(cidx_v1, tok_v1, semc1)
    fire_idx(k + 2, idx_v1, cidx_v1, semi1)
    fire_out(k, tok_v1, semo1)

    drain_g(idx_v0, tok_v0, semg0)
    drain_out(tok_v1, semo1)
    drain_idx(idx_v1, cidx_v1, semi1)
    fire_g(idx_v1, tok_v1, semg1)
    fire_comb(cidx_v0, tok_v0, semc0)
    drain_comb(cidx_v0, tok_v0, semc0)

    @pl.when(k + 3 < n_chunks)
    def _():
      fire_idx(k + 3, idx_v0, cidx_v0, semi0)

    fire_out(k + 1, tok_v0, semo0)

  drain_g(idx_v1, tok_v1, semg1)
  drain_out(tok_v0, semo0)
  fire_comb(cidx_v1, tok_v1, semc1)
  drain_comb(cidx_v1, tok_v1, semc1)
  fire_out(n_chunks - 1, tok_v1, semo1)
  drain_out(tok_v1, semo1)


def kernel(x, segment_label, token_table, position_table, segment_table):
  batch, seq = x.shape
  emb = token_table.shape[1]
  n = batch * seq

  table128 = jnp.pad(token_table, ((0, 0), (0, 128 - emb)))

  nseg = segment_table.shape[0]
  combined = (position_table[:seq, None, :]
              + segment_table[None, :, :]).reshape(seq * nseg, emb)
  comb128 = jnp.pad(combined, ((0, 0), (0, 128 - emb)))

  idx = x.reshape(n).astype(jnp.int32)
  cidx = (jnp.arange(seq, dtype=jnp.int32)[None, :] * nseg
          + segment_label.astype(jnp.int32)).reshape(n)

  mesh = plsc.VectorSubcoreMesh(core_axis_name="c", subcore_axis_name="s",
                                num_cores=NC, num_subcores=NS)
  run = pl.kernel(
      _emb_kernel,
      out_type=jax.ShapeDtypeStruct((n, 128), jnp.float32),
      mesh=mesh,
      scratch_types=[
          pltpu.VMEM((CHUNK,), jnp.int32),
          pltpu.VMEM((CHUNK,), jnp.int32),
          pltpu.VMEM((CHUNK,), jnp.int32),
          pltpu.VMEM((CHUNK,), jnp.int32),
          pltpu.VMEM((CHUNK, 128), jnp.float32),
          pltpu.VMEM((CHUNK, 128), jnp.float32),
          pltpu.VMEM_SHARED((seq * segment_table.shape[0], 128), jnp.float32),
          pltpu.SemaphoreType.DMA,
          pltpu.SemaphoreType.DMA,
          pltpu.SemaphoreType.DMA,
          pltpu.SemaphoreType.DMA,
          pltpu.SemaphoreType.DMA,
          pltpu.SemaphoreType.DMA,
          pltpu.SemaphoreType.DMA,
          pltpu.SemaphoreType.DMA,
      ],
      compiler_params=pltpu.CompilerParams(use_tc_tiling_on_sc=True),
  )
  out128 = run(table128, comb128, idx, cidx)
  return out128.reshape(batch, seq, 128)[:, :, :emb]


# speedup vs baseline: 3.6461x; 1.0006x over previous
import jax
import jax.numpy as jnp
from jax import lax
from jax.experimental import pallas as pl
from jax.experimental.pallas import tpu as pltpu
from jax.experimental.pallas import tpu_sc as plsc

NC = 2
NS = 16
NW = NC * NS
CHUNK = 320
WINDOWS = ((0, 128), (128, 128), (256, 64))


def _emb_kernel(tok_hbm, comb_hbm, idx_hbm, cidx_hbm, out_hbm,
                idx_v0, idx_v1, cidx_v0, cidx_v1, tok_v0, tok_v1, comb_sh,
                semg0, semg1, semc0, semc1, semo0, semo1, semi0, semi1):
  n_rows = out_hbm.shape[0]
  rows_per_tile = n_rows // NW
  n_chunks = rows_per_tile // CHUNK
  wid = lax.axis_index("s") * NC + lax.axis_index("c")
  row0 = wid * rows_per_tile

  @pl.when(lax.axis_index("s") == 0)
  def _load_comb():
    pltpu.sync_copy(comb_hbm, comb_sh)

  plsc.subcore_barrier()

  def fire_idx(k, iv, cv, sem):
    rb = row0 + k * CHUNK
    pltpu.async_copy(idx_hbm.at[pl.ds(rb, CHUNK)], iv, sem)
    pltpu.async_copy(cidx_hbm.at[pl.ds(rb, CHUNK)], cv, sem)

  def drain_idx(iv, cv, sem):
    pltpu.make_async_copy(idx_hbm.at[pl.ds(0, CHUNK)], iv, sem).wait()
    pltpu.make_async_copy(cidx_hbm.at[pl.ds(0, CHUNK)], cv, sem).wait()

  def fire_g(iv, tv, sem):
    for off, ln in WINDOWS:
      w = pl.ds(off, ln)
      pltpu.async_copy(tok_hbm.at[iv.at[w]], tv.at[w], sem)

  def drain_g(iv, tv, sem):
    for off, ln in WINDOWS:
      w = pl.ds(off, ln)
      pltpu.make_async_copy(tok_hbm.at[iv.at[w]], tv.at[w], sem).wait()

  def fire_comb(cv, tv, sem):
    for off, ln in WINDOWS:
      w = pl.ds(off, ln)
      pltpu.async_copy(comb_sh.at[cv.at[w]], tv.at[w], sem, add=True)

  def drain_comb(cv, tv, sem):
    for off, ln in WINDOWS:
      w = pl.ds(off, ln)
      pltpu.make_async_copy(comb_sh.at[cv.at[w]], tv.at[w], sem).wait()

  def fire_out(k, tv, sem):
    rb = row0 + k * CHUNK
    pltpu.async_copy(tv, out_hbm.at[pl.ds(rb, CHUNK)], sem)

  def drain_out(tv, sem):
    pltpu.make_async_copy(tv, out_hbm.at[pl.ds(0, CHUNK)], sem).wait()

  fire_idx(0, idx_v0, cidx_v0, semi0)
  fire_idx(1, idx_v1, cidx_v1, semi1)
  drain_idx(idx_v0, cidx_v0, semi0)
  fire_g(idx_v0, tok_v0, semg0)

  drain_g(idx_v0, tok_v0, semg0)
  drain_idx(idx_v1, cidx_v1, semi1)
  fire_g(idx_v1, tok_v1, semg1)
  fire_comb(cidx_v0, tok_v0, semc0)
  drain_comb(cidx_v0, tok_v0, semc0)
  fire_idx(2, idx_v0, cidx_v0, semi0)
  fire_out(0, tok_v0, semo0)

  @pl.loop(1, n_chunks - 2, step=2)
  def _pair(k):
    drain_g(idx_v1, tok_v1, semg1)
    drain_out(tok_v0, semo0)
    drain_idx(idx_v0, cidx_v0, semi0)
    fire_g(idx_v0, tok_v0, semg0)
    fire_comb(cidx_v1, tok_v1, semc1)
    drain_comb(cidx_v1, tok_v1, semc1)
    fire_idx(k + 2, idx_v1, cidx_v1, semi1)
    fire_out(k, tok_v1, semo1)

    drain_g(idx_v0, tok_v0, semg0)
    drain_out(tok_v1, semo1)
    drain_idx(idx_v1, cidx_v1, semi1)
    fire_g(idx_v1, tok_v1, semg1)
    fire_comb(cidx_v0, tok_v0, semc0)
    drain_comb(cidx_v0, tok_v0, semc0)

    @pl.when(k + 3 < n_chunks)
    def _():
      fire_idx(k + 3, idx_v0, cidx_v0, semi0)

    fire_out(k + 1, tok_v0, semo0)

  drain_g(idx_v1, tok_v1, semg1)
  drain_out(tok_v0, semo0)
  fire_comb(cidx_v1, tok_v1, semc1)
  drain_comb(cidx_v1, tok_v1, semc1)
  fire_out(n_chunks - 1, tok_v1, semo1)
  drain_out(tok_v1, semo1)


def kernel(x, segment_label, token_table, position_table, segment_table):
  batch, seq = x.shape
  emb = token_table.shape[1]
  n = batch * seq

  table128 = jnp.pad(token_table, ((0, 0), (0, 128 - emb)))

  nseg = segment_table.shape[0]
  combined = (position_table[:seq, None, :]
              + segment_table[None, :, :]).reshape(seq * nseg, emb)
  comb128 = jnp.pad(combined, ((0, 0), (0, 128 - emb)))

  idx = x.reshape(n).astype(jnp.int32)
  cidx = (jnp.arange(seq, dtype=jnp.int32)[None, :] * nseg
          + segment_label.astype(jnp.int32)).reshape(n)

  mesh = plsc.VectorSubcoreMesh(core_axis_name="c", subcore_axis_name="s",
                                num_cores=NC, num_subcores=NS)
  run = pl.kernel(
      _emb_kernel,
      out_type=jax.ShapeDtypeStruct((n, 128), jnp.float32),
      mesh=mesh,
      scratch_types=[
          pltpu.VMEM((CHUNK,), jnp.int32),
          pltpu.VMEM((CHUNK,), jnp.int32),
          pltpu.VMEM((CHUNK,), jnp.int32),
          pltpu.VMEM((CHUNK,), jnp.int32),
          pltpu.VMEM((CHUNK, 128), jnp.float32),
          pltpu.VMEM((CHUNK, 128), jnp.float32),
          pltpu.VMEM_SHARED((seq * segment_table.shape[0], 128), jnp.float32),
          pltpu.SemaphoreType.DMA,
          pltpu.SemaphoreType.DMA,
          pltpu.SemaphoreType.DMA,
          pltpu.SemaphoreType.DMA,
          pltpu.SemaphoreType.DMA,
          pltpu.SemaphoreType.DMA,
          pltpu.SemaphoreType.DMA,
          pltpu.SemaphoreType.DMA,
      ],
      compiler_params=pltpu.CompilerParams(use_tc_tiling_on_sc=True),
  )
  out128 = run(table128, comb128, idx, cidx)
  return out128.reshape(batch, seq, 128)[:, :, :emb]


# speedup vs baseline: 3.6509x; 1.0013x over previous
import jax
import jax.numpy as jnp
from jax import lax
from jax.experimental import pallas as pl
from jax.experimental.pallas import tpu as pltpu
from jax.experimental.pallas import tpu_sc as plsc

NC = 2
NS = 16
NW = NC * NS
CHUNK = 320
WINDOWS = ((0, 128), (128, 128), (256, 64))


def _emb_kernel(tok_hbm, comb_hbm, idx_hbm, cidx_hbm, out_hbm,
                idx_v0, idx_v1, cidx_v0, cidx_v1, tok_v0, tok_v1, comb_sh,
                semg0, semg1, semc0, semc1, semo0, semo1, semi0, semi1):
  n_rows = out_hbm.shape[0]
  rows_per_tile = n_rows // NW
  n_chunks = rows_per_tile // CHUNK
  wid = lax.axis_index("s") * NC + lax.axis_index("c")
  row0 = wid * rows_per_tile

  @pl.when(lax.axis_index("s") == 0)
  def _load_comb():
    pltpu.sync_copy(comb_hbm, comb_sh)

  plsc.subcore_barrier()

  def fire_idx(k, iv, cv, sem):
    rb = row0 + k * CHUNK
    pltpu.async_copy(idx_hbm.at[pl.ds(rb, CHUNK)], iv, sem)
    pltpu.async_copy(cidx_hbm.at[pl.ds(rb, CHUNK)], cv, sem)

  def drain_idx(iv, cv, sem):
    pltpu.make_async_copy(idx_hbm.at[pl.ds(0, CHUNK)], iv, sem).wait()
    pltpu.make_async_copy(cidx_hbm.at[pl.ds(0, CHUNK)], cv, sem).wait()

  def fire_g(iv, tv, sem):
    for off, ln in WINDOWS:
      w = pl.ds(off, ln)
      pltpu.async_copy(tok_hbm.at[iv.at[w]], tv.at[w], sem)

  def drain_g(iv, tv, sem):
    for off, ln in WINDOWS:
      w = pl.ds(off, ln)
      pltpu.make_async_copy(tok_hbm.at[iv.at[w]], tv.at[w], sem).wait()

  def fire_comb(cv, tv, sem):
    for off, ln in WINDOWS:
      w = pl.ds(off, ln)
      pltpu.async_copy(comb_sh.at[cv.at[w]], tv.at[w], sem, add=True)

  def drain_comb(cv, tv, sem):
    for off, ln in WINDOWS:
      w = pl.ds(off, ln)
      pltpu.make_async_copy(comb_sh.at[cv.at[w]], tv.at[w], sem).wait()

  def fire_out(k, tv, sem):
    rb = row0 + k * CHUNK
    pltpu.async_copy(tv, out_hbm.at[pl.ds(rb, CHUNK)], sem)

  def drain_out(tv, sem):
    pltpu.make_async_copy(tv, out_hbm.at[pl.ds(0, CHUNK)], sem).wait()

  fire_idx(0, idx_v0, cidx_v0, semi0)
  fire_idx(1, idx_v1, cidx_v1, semi1)
  drain_idx(idx_v0, cidx_v0, semi0)
  fire_g(idx_v0, tok_v0, semg0)

  drain_g(idx_v0, tok_v0, semg0)
  drain_idx(idx_v1, cidx_v1, semi1)
  fire_g(idx_v1, tok_v1, semg1)
  fire_comb(cidx_v0, tok_v0, semc0)
  drain_comb(cidx_v0, tok_v0, semc0)
  fire_idx(2, idx_v0, cidx_v0, semi0)
  fire_out(0, tok_v0, semo0)

  @pl.loop(1, n_chunks - 2, step=2)
  def _pair(k):
    drain_g(idx_v1, tok_v1, semg1)
    drain_out(tok_v0, semo0)
    drain_idx(idx_v0, cidx_v0, semi0)
    fire_g(idx_v0, tok_v0, semg0)
    fire_comb(cidx_v1, tok_v1, semc1)
    drain_comb(cidx_v1, tok_v1, semc1)
    fire_idx(k + 2, idx_v1, cidx_v1, semi1)
    fire_out(k, tok_v1, semo1)

    drain_g(idx_v0, tok_v0, semg0)
    drain_out(tok_v1, semo1)
    drain_idx(idx_v1, cidx_v1, semi1)
    fire_g(idx_v1, tok_v1, semg1)
    fire_comb(cidx_v0, tok_v0, semc0)
    drain_comb(cidx_v0, tok_v0, semc0)

    @pl.when(k + 3 < n_chunks)
    def _():
      fire_idx(k + 3, idx_v0, cidx_v0, semi0)

    fire_out(k + 1, tok_v0, semo0)

  drain_g(idx_v1, tok_v1, semg1)
  drain_out(tok_v0, semo0)
  fire_comb(cidx_v1, tok_v1, semc1)
  drain_comb(cidx_v1, tok_v1, semc1)
  fire_out(n_chunks - 1, tok_v1, semo1)
  drain_out(tok_v1, semo1)


def kernel(x, segment_label, token_table, position_table, segment_table):
  batch, seq = x.shape
  emb = token_table.shape[1]
  n = batch * seq

  table128 = jnp.pad(token_table.T, ((0, 128 - emb), (0, 0))).T

  nseg = segment_table.shape[0]
  combined = (position_table[:seq, None, :]
              + segment_table[None, :, :]).reshape(seq * nseg, emb)
  comb128 = jnp.pad(combined, ((0, 0), (0, 128 - emb)))

  idx = x.reshape(n).astype(jnp.int32)
  cidx = (jnp.arange(seq, dtype=jnp.int32)[None, :] * nseg
          + segment_label.astype(jnp.int32)).reshape(n)

  mesh = plsc.VectorSubcoreMesh(core_axis_name="c", subcore_axis_name="s",
                                num_cores=NC, num_subcores=NS)
  run = pl.kernel(
      _emb_kernel,
      out_type=jax.ShapeDtypeStruct((n, 128), jnp.float32),
      mesh=mesh,
      scratch_types=[
          pltpu.VMEM((CHUNK,), jnp.int32),
          pltpu.VMEM((CHUNK,), jnp.int32),
          pltpu.VMEM((CHUNK,), jnp.int32),
          pltpu.VMEM((CHUNK,), jnp.int32),
          pltpu.VMEM((CHUNK, 128), jnp.float32),
          pltpu.VMEM((CHUNK, 128), jnp.float32),
          pltpu.VMEM_SHARED((seq * segment_table.shape[0], 128), jnp.float32),
          pltpu.SemaphoreType.DMA,
          pltpu.SemaphoreType.DMA,
          pltpu.SemaphoreType.DMA,
          pltpu.SemaphoreType.DMA,
          pltpu.SemaphoreType.DMA,
          pltpu.SemaphoreType.DMA,
          pltpu.SemaphoreType.DMA,
          pltpu.SemaphoreType.DMA,
      ],
      compiler_params=pltpu.CompilerParams(use_tc_tiling_on_sc=True),
  )
  out128 = run(table128, comb128, idx, cidx)
  return out128.reshape(batch, seq, 128)[:, :, :emb]
